# Initial kernel scaffold; baseline (speedup 1.0000x reference)
import jax
import jax.numpy as jnp
from jax.experimental import pallas as pl


def kernel(im_q, im_k, batch, bag_idx, label, bag_label, W_enc_q, W_self_q, V_q, U_q, w_att_q, W_cls_q, W_enc_k, W_self_k, bank):
    raise NotImplementedError("write your pallas kernel here")


# speedup vs baseline: 4.1778x; 4.1778x over previous
import jax
import jax.numpy as jnp
from jax import lax
from jax.experimental import pallas as pl

N_INST = 8192
D_IN = 1024
DIM = 128
B = 128
K = 65536
T = 0.07
EMA = 0.999

ROWS = 512
COLS = 2048


def _enc_body(imq_ref, imk_ref, wq_ref, wk_ref, v_ref, u_ref, wa_ref,
              feaq_ref, sfq_ref, feak_ref, sfk_ref, sq_ref, sk_ref):
    wq = wq_ref[...]
    wk = EMA * wk_ref[...] + (1.0 - EMA) * wq
    hq = jnp.dot(imq_ref[...], wq, preferred_element_type=jnp.float32)
    hk = jnp.dot(imk_ref[...], wk, preferred_element_type=jnp.float32)
    feaq = jnp.tanh(hq[:, :DIM])
    sfq = jnp.tanh(hq[:, DIM:])
    feak = jnp.tanh(hk[:, :DIM])
    sfk = jnp.tanh(hk[:, DIM:])
    feaq_ref[...] = feaq
    sfq_ref[...] = sfq
    feak_ref[...] = feak
    sfk_ref[...] = sfk
    v = v_ref[...]
    u = u_ref[...]
    wa = wa_ref[...]
    aq = jnp.tanh(jnp.dot(feaq, v, preferred_element_type=jnp.float32)) * \
        jax.nn.sigmoid(jnp.dot(feaq, u, preferred_element_type=jnp.float32))
    ak = jnp.tanh(jnp.dot(feak, v, preferred_element_type=jnp.float32)) * \
        jax.nn.sigmoid(jnp.dot(feak, u, preferred_element_type=jnp.float32))
    sq_ref[...] = jnp.dot(aq, wa, preferred_element_type=jnp.float32)
    sk_ref[...] = jnp.dot(ak, wa, preferred_element_type=jnp.float32)


def _agg_body(feaq_ref, feak_ref, sq_ref, sk_ref, batch_ref, wcls_ref,
              attq_ref, attk_ref, yprob_ref, q_ref, k_ref, lpos_ref):
    batch = batch_ref[...]
    seg = lax.broadcasted_iota(jnp.int32, (1, B), 1)
    onehot_b = batch == seg
    onehot = onehot_b.astype(jnp.float32)

    def branch(fea, s):
        sm = jnp.max(jnp.where(onehot_b, s, -1e30), axis=0, keepdims=True)
        sm = jnp.where(sm > -1e29, sm, 0.0)
        srow = jnp.sum(onehot * sm, axis=1, keepdims=True)
        e = jnp.exp(s - srow)
        denom = lax.dot_general(e, onehot, (((0,), (0,)), ((), ())),
                                preferred_element_type=jnp.float32)
        drow = jnp.sum(onehot * denom, axis=1, keepdims=True)
        att = e / (drow + 1e-9)
        bagf = lax.dot_general(onehot, att * fea, (((0,), (0,)), ((), ())),
                               preferred_element_type=jnp.float32)
        nrm = jnp.sqrt(jnp.sum(bagf * bagf, axis=1, keepdims=True))
        return att, bagf, bagf / (nrm + 1e-12)

    attq, bagfq, qn = branch(feaq_ref[...], sq_ref[...])
    attk, _, kn = branch(feak_ref[...], sk_ref[...])
    attq_ref[...] = attq
    attk_ref[...] = attk
    yprob_ref[...] = jax.nn.sigmoid(
        jnp.dot(bagfq, wcls_ref[...], preferred_element_type=jnp.float32))
    q_ref[...] = qn
    k_ref[...] = kn
    lpos_ref[...] = jnp.sum(qn * kn, axis=1, keepdims=True)


def _bank_body(q_ref, lab_ref, bl_ref, bic_ref, bir_ref, bank_ref,
               lneg_ref, nbank_ref):
    j = pl.program_id(0)
    qm = q_ref[...]
    bank_t = bank_ref[...]
    ln = jnp.dot(qm, bank_t, preferred_element_type=jnp.float32)
    bl = bl_ref[0]
    mask = lab_ref[...] == bl
    lneg_ref[...] = jnp.where(mask, -1e9, ln) / T
    bic = bic_ref[...]
    bir = bir_ref[...]
    ir = lax.broadcasted_iota(jnp.int32, (1, B), 1)
    ic = lax.broadcasted_iota(jnp.int32, (B, 1), 0)
    dup_later = (bic == bir) & (ir > ic)
    is_last = jnp.max(dup_later.astype(jnp.int32), axis=1, keepdims=True) == 0
    cols = lax.broadcasted_iota(jnp.int32, (B, COLS), 1) + j * COLS
    sel = ((bic == cols) & is_last).astype(jnp.float32)
    hit = jnp.max(sel, axis=0, keepdims=True)
    over = lax.dot_general(qm, sel, (((0,), (0,)), ((), ())),
                           preferred_element_type=jnp.float32)
    nbank_ref[...] = bank_t * (1.0 - hit) + over


def kernel(im_q, im_k, batch, bag_idx, label, bag_label, W_enc_q, W_self_q,
           V_q, U_q, w_att_q, W_cls_q, W_enc_k, W_self_k, bank):
    f32 = jnp.float32
    wq_cat = jnp.concatenate([W_enc_q, W_self_q], axis=1)
    wk_cat = jnp.concatenate([W_enc_k, W_self_k], axis=1)

    n_row_blocks = N_INST // ROWS
    feaq, sfq, feak, sfk, sq, sk = pl.pallas_call(
        _enc_body,
        grid=(n_row_blocks,),
        in_specs=[
            pl.BlockSpec((ROWS, D_IN), lambda i: (i, 0)),
            pl.BlockSpec((ROWS, D_IN), lambda i: (i, 0)),
            pl.BlockSpec((D_IN, 2 * DIM), lambda i: (0, 0)),
            pl.BlockSpec((D_IN, 2 * DIM), lambda i: (0, 0)),
            pl.BlockSpec((DIM, DIM), lambda i: (0, 0)),
            pl.BlockSpec((DIM, DIM), lambda i: (0, 0)),
            pl.BlockSpec((DIM, 1), lambda i: (0, 0)),
        ],
        out_specs=[
            pl.BlockSpec((ROWS, DIM), lambda i: (i, 0)),
            pl.BlockSpec((ROWS, DIM), lambda i: (i, 0)),
            pl.BlockSpec((ROWS, DIM), lambda i: (i, 0)),
            pl.BlockSpec((ROWS, DIM), lambda i: (i, 0)),
            pl.BlockSpec((ROWS, 1), lambda i: (i, 0)),
            pl.BlockSpec((ROWS, 1), lambda i: (i, 0)),
        ],
        out_shape=[
            jax.ShapeDtypeStruct((N_INST, DIM), f32),
            jax.ShapeDtypeStruct((N_INST, DIM), f32),
            jax.ShapeDtypeStruct((N_INST, DIM), f32),
            jax.ShapeDtypeStruct((N_INST, DIM), f32),
            jax.ShapeDtypeStruct((N_INST, 1), f32),
            jax.ShapeDtypeStruct((N_INST, 1), f32),
        ],
    )(im_q, im_k, wq_cat, wk_cat, V_q, U_q, w_att_q)

    attq, attk, yprob, qn, kn, lpos = pl.pallas_call(
        _agg_body,
        out_shape=[
            jax.ShapeDtypeStruct((N_INST, 1), f32),
            jax.ShapeDtypeStruct((N_INST, 1), f32),
            jax.ShapeDtypeStruct((B, 1), f32),
            jax.ShapeDtypeStruct((B, DIM), f32),
            jax.ShapeDtypeStruct((B, DIM), f32),
            jax.ShapeDtypeStruct((B, 1), f32),
        ],
    )(feaq, feak, sq, sk, batch.reshape(N_INST, 1).astype(jnp.int32),
      W_cls_q)

    n_col_blocks = K // COLS
    lneg, nbank = pl.pallas_call(
        _bank_body,
        grid=(n_col_blocks,),
        in_specs=[
            pl.BlockSpec((B, DIM), lambda j: (0, 0)),
            pl.BlockSpec((B, 1), lambda j: (0, 0)),
            pl.BlockSpec((1, 1, COLS), lambda j: (j, 0, 0)),
            pl.BlockSpec((B, 1), lambda j: (0, 0)),
            pl.BlockSpec((1, B), lambda j: (0, 0)),
            pl.BlockSpec((DIM, COLS), lambda j: (0, j)),
        ],
        out_specs=[
            pl.BlockSpec((B, COLS), lambda j: (0, j)),
            pl.BlockSpec((DIM, COLS), lambda j: (0, j)),
        ],
        out_shape=[
            jax.ShapeDtypeStruct((B, K), f32),
            jax.ShapeDtypeStruct((DIM, K), f32),
        ],
    )(qn, label.reshape(B, 1).astype(jnp.int32),
      bag_label.reshape(n_col_blocks, 1, COLS).astype(jnp.int32),
      bag_idx.reshape(B, 1).astype(jnp.int32),
      bag_idx.reshape(1, B).astype(jnp.int32), bank)

    logits = jnp.concatenate([lpos / T, lneg], axis=1)
    labels = jnp.zeros((B,), jnp.int32)
    return (yprob, logits, labels, nbank, sfq, sfk,
            attq.reshape(N_INST), attk.reshape(N_INST),
            sq.reshape(N_INST), sk.reshape(N_INST))


# speedup vs baseline: 4.9107x; 1.1754x over previous
import jax
import jax.numpy as jnp
from jax import lax
from jax.experimental import pallas as pl
from jax.experimental.pallas import tpu as pltpu

N_INST = 8192
D_IN = 1024
DIM = 128
B = 128
K = 65536
T = 0.07
EMA = 0.999

ROWS = 512
COLS = 2048


def _enc_body(imq_ref, imk_ref, wq_ref, wk_ref, v_ref, u_ref, wa_ref,
              feaq_ref, sfq_ref, feak_ref, sfk_ref, sq_ref, sk_ref):
    wq = wq_ref[...]
    wk = EMA * wk_ref[...] + (1.0 - EMA) * wq
    hq = jnp.dot(imq_ref[...], wq, preferred_element_type=jnp.float32)
    hk = jnp.dot(imk_ref[...], wk, preferred_element_type=jnp.float32)
    feaq = jnp.tanh(hq[:, :DIM])
    sfq = jnp.tanh(hq[:, DIM:])
    feak = jnp.tanh(hk[:, :DIM])
    sfk = jnp.tanh(hk[:, DIM:])
    feaq_ref[...] = feaq
    sfq_ref[...] = sfq
    feak_ref[...] = feak
    sfk_ref[...] = sfk
    v = v_ref[...]
    u = u_ref[...]
    wa = wa_ref[...]
    aq = jnp.tanh(jnp.dot(feaq, v, preferred_element_type=jnp.float32)) * \
        jax.nn.sigmoid(jnp.dot(feaq, u, preferred_element_type=jnp.float32))
    ak = jnp.tanh(jnp.dot(feak, v, preferred_element_type=jnp.float32)) * \
        jax.nn.sigmoid(jnp.dot(feak, u, preferred_element_type=jnp.float32))
    sq_ref[...] = jnp.dot(aq, wa, preferred_element_type=jnp.float32)
    sk_ref[...] = jnp.dot(ak, wa, preferred_element_type=jnp.float32)


def _agg_body(feaq_ref, feak_ref, sq_ref, sk_ref, batch_ref, wcls_ref,
              attq_ref, attk_ref, yprob_ref, q_ref, k_ref, lpos_ref):
    batch = batch_ref[...]
    seg = lax.broadcasted_iota(jnp.int32, (1, B), 1)
    onehot_b = batch == seg
    onehot = onehot_b.astype(jnp.float32)

    def branch(fea, s):
        sm = jnp.max(jnp.where(onehot_b, s, -1e30), axis=0, keepdims=True)
        sm = jnp.where(sm > -1e29, sm, 0.0)
        srow = jnp.sum(onehot * sm, axis=1, keepdims=True)
        e = jnp.exp(s - srow)
        denom = lax.dot_general(e, onehot, (((0,), (0,)), ((), ())),
                                preferred_element_type=jnp.float32)
        drow = jnp.sum(onehot * denom, axis=1, keepdims=True)
        att = e / (drow + 1e-9)
        bagf = lax.dot_general(onehot, att * fea, (((0,), (0,)), ((), ())),
                               preferred_element_type=jnp.float32)
        nrm = jnp.sqrt(jnp.sum(bagf * bagf, axis=1, keepdims=True))
        return att, bagf, bagf / (nrm + 1e-12)

    attq, bagfq, qn = branch(feaq_ref[...], sq_ref[...])
    attk, _, kn = branch(feak_ref[...], sk_ref[...])
    attq_ref[...] = attq
    attk_ref[...] = attk
    yprob_ref[...] = jax.nn.sigmoid(
        jnp.dot(bagfq, wcls_ref[...], preferred_element_type=jnp.float32))
    q_ref[...] = qn
    k_ref[...] = kn
    lpos_ref[...] = jnp.sum(qn * kn, axis=1, keepdims=True) / T


def _bank_body(q_ref, lpos_ref, lab_ref, bl_ref, bic_ref, bir_ref, bank_ref,
               logits_ref, nbank_ref, carry):
    j = pl.program_id(0)
    nb = K // COLS

    @pl.when(j < nb)
    def _():
        qm = q_ref[...]
        bank_t = bank_ref[...]
        ln = jnp.dot(qm, bank_t, preferred_element_type=jnp.float32)
        bl = bl_ref[0]
        mask = lab_ref[...] == bl
        ln = jnp.where(mask, -1e9, ln) / T
        head = jnp.where(j == 0, lpos_ref[...], carry[...])
        logits_ref[...] = jnp.concatenate([head, ln[:, :COLS - 1]], axis=1)
        carry[...] = ln[:, COLS - 1:COLS]
        bic = bic_ref[...]
        bir = bir_ref[...]
        ir = lax.broadcasted_iota(jnp.int32, (1, B), 1)
        ic = lax.broadcasted_iota(jnp.int32, (B, 1), 0)
        dup_later = (bic == bir) & (ir > ic)
        is_last = jnp.max(dup_later.astype(jnp.int32), axis=1,
                          keepdims=True) == 0
        cols = lax.broadcasted_iota(jnp.int32, (B, COLS), 1) + j * COLS
        sel = ((bic == cols) & is_last).astype(jnp.float32)
        hit = jnp.max(sel, axis=0, keepdims=True)
        over = lax.dot_general(qm, sel, (((0,), (0,)), ((), ())),
                               preferred_element_type=jnp.float32)
        nbank_ref[...] = bank_t * (1.0 - hit) + over

    @pl.when(j == nb)
    def _():
        logits_ref[:, 0:1] = carry[...]


def kernel(im_q, im_k, batch, bag_idx, label, bag_label, W_enc_q, W_self_q,
           V_q, U_q, w_att_q, W_cls_q, W_enc_k, W_self_k, bank):
    f32 = jnp.float32
    wq_cat = jnp.concatenate([W_enc_q, W_self_q], axis=1)
    wk_cat = jnp.concatenate([W_enc_k, W_self_k], axis=1)

    n_row_blocks = N_INST // ROWS
    feaq, sfq, feak, sfk, sq, sk = pl.pallas_call(
        _enc_body,
        grid=(n_row_blocks,),
        in_specs=[
            pl.BlockSpec((ROWS, D_IN), lambda i: (i, 0)),
            pl.BlockSpec((ROWS, D_IN), lambda i: (i, 0)),
            pl.BlockSpec((D_IN, 2 * DIM), lambda i: (0, 0)),
            pl.BlockSpec((D_IN, 2 * DIM), lambda i: (0, 0)),
            pl.BlockSpec((DIM, DIM), lambda i: (0, 0)),
            pl.BlockSpec((DIM, DIM), lambda i: (0, 0)),
            pl.BlockSpec((DIM, 1), lambda i: (0, 0)),
        ],
        out_specs=[
            pl.BlockSpec((ROWS, DIM), lambda i: (i, 0)),
            pl.BlockSpec((ROWS, DIM), lambda i: (i, 0)),
            pl.BlockSpec((ROWS, DIM), lambda i: (i, 0)),
            pl.BlockSpec((ROWS, DIM), lambda i: (i, 0)),
            pl.BlockSpec((ROWS, 1), lambda i: (i, 0)),
            pl.BlockSpec((ROWS, 1), lambda i: (i, 0)),
        ],
        out_shape=[
            jax.ShapeDtypeStruct((N_INST, DIM), f32),
            jax.ShapeDtypeStruct((N_INST, DIM), f32),
            jax.ShapeDtypeStruct((N_INST, DIM), f32),
            jax.ShapeDtypeStruct((N_INST, DIM), f32),
            jax.ShapeDtypeStruct((N_INST, 1), f32),
            jax.ShapeDtypeStruct((N_INST, 1), f32),
        ],
    )(im_q, im_k, wq_cat, wk_cat, V_q, U_q, w_att_q)

    attq, attk, yprob, qn, kn, lpos = pl.pallas_call(
        _agg_body,
        out_shape=[
            jax.ShapeDtypeStruct((N_INST, 1), f32),
            jax.ShapeDtypeStruct((N_INST, 1), f32),
            jax.ShapeDtypeStruct((B, 1), f32),
            jax.ShapeDtypeStruct((B, DIM), f32),
            jax.ShapeDtypeStruct((B, DIM), f32),
            jax.ShapeDtypeStruct((B, 1), f32),
        ],
    )(feaq, feak, sq, sk, batch.reshape(N_INST, 1).astype(jnp.int32),
      W_cls_q)

    n_col_blocks = K // COLS
    last = n_col_blocks - 1
    logits, nbank = pl.pallas_call(
        _bank_body,
        grid=(n_col_blocks + 1,),
        in_specs=[
            pl.BlockSpec((B, DIM), lambda j: (0, 0)),
            pl.BlockSpec((B, 1), lambda j: (0, 0)),
            pl.BlockSpec((B, 1), lambda j: (0, 0)),
            pl.BlockSpec((1, 1, COLS), lambda j: (jnp.minimum(j, last), 0, 0)),
            pl.BlockSpec((B, 1), lambda j: (0, 0)),
            pl.BlockSpec((1, B), lambda j: (0, 0)),
            pl.BlockSpec((DIM, COLS), lambda j: (0, jnp.minimum(j, last))),
        ],
        out_specs=[
            pl.BlockSpec((B, COLS), lambda j: (0, j)),
            pl.BlockSpec((DIM, COLS), lambda j: (0, jnp.minimum(j, last))),
        ],
        out_shape=[
            jax.ShapeDtypeStruct((B, K + 1), f32),
            jax.ShapeDtypeStruct((DIM, K), f32),
        ],
        scratch_shapes=[
            pltpu.VMEM((B, 1), f32),
        ],
    )(qn, lpos, label.reshape(B, 1).astype(jnp.int32),
      bag_label.reshape(n_col_blocks, 1, COLS).astype(jnp.int32),
      bag_idx.reshape(B, 1).astype(jnp.int32),
      bag_idx.reshape(1, B).astype(jnp.int32), bank)

    labels = jnp.zeros((B,), jnp.int32)
    return (yprob, logits, labels, nbank, sfq, sfk,
            attq.reshape(N_INST), attk.reshape(N_INST),
            sq.reshape(N_INST), sk.reshape(N_INST))


# speedup vs baseline: 4.9773x; 1.0136x over previous
import jax
import jax.numpy as jnp
from jax import lax
from jax.experimental import pallas as pl
from jax.experimental.pallas import tpu as pltpu

N_INST = 8192
D_IN = 1024
DIM = 128
B = 128
K = 65536
T = 0.07
EMA = 0.999

ROWS = 512
COLS = 2048


def _enc_agg_body(imq_ref, imk_ref, wq_ref, wk_ref, v_ref, u_ref, wa_ref,
                  batch_ref, wcls_ref,
                  sfq_ref, sfk_ref, sqo_ref, sko_ref,
                  attq_ref, attk_ref, yprob_ref, q_ref, k_ref, lpos_ref,
                  feaq_s, feak_s, sq_s, sk_s):
    i = pl.program_id(0)
    wq = wq_ref[...]
    wk = EMA * wk_ref[...] + (1.0 - EMA) * wq
    hq = jnp.dot(imq_ref[...], wq, preferred_element_type=jnp.float32)
    hk = jnp.dot(imk_ref[...], wk, preferred_element_type=jnp.float32)
    feaq = jnp.tanh(hq[:, :DIM])
    sfq = jnp.tanh(hq[:, DIM:])
    feak = jnp.tanh(hk[:, :DIM])
    sfk = jnp.tanh(hk[:, DIM:])
    sfq_ref[...] = sfq
    sfk_ref[...] = sfk
    v = v_ref[...]
    u = u_ref[...]
    wa = wa_ref[...]
    aq = jnp.tanh(jnp.dot(feaq, v, preferred_element_type=jnp.float32)) * \
        jax.nn.sigmoid(jnp.dot(feaq, u, preferred_element_type=jnp.float32))
    ak = jnp.tanh(jnp.dot(feak, v, preferred_element_type=jnp.float32)) * \
        jax.nn.sigmoid(jnp.dot(feak, u, preferred_element_type=jnp.float32))
    sq = jnp.dot(aq, wa, preferred_element_type=jnp.float32)
    sk = jnp.dot(ak, wa, preferred_element_type=jnp.float32)
    sqo_ref[...] = sq
    sko_ref[...] = sk
    base = i * ROWS
    feaq_s[pl.ds(base, ROWS), :] = feaq
    feak_s[pl.ds(base, ROWS), :] = feak
    sq_s[pl.ds(base, ROWS), :] = sq
    sk_s[pl.ds(base, ROWS), :] = sk

    @pl.when(i == (N_INST // ROWS) - 1)
    def _():
        batch = batch_ref[...]
        seg = lax.broadcasted_iota(jnp.int32, (1, B), 1)
        onehot_b = batch == seg
        onehot = onehot_b.astype(jnp.float32)

        def branch(fea, s):
            sm = jnp.max(jnp.where(onehot_b, s, -1e30), axis=0, keepdims=True)
            sm = jnp.where(sm > -1e29, sm, 0.0)
            srow = jnp.sum(onehot * sm, axis=1, keepdims=True)
            e = jnp.exp(s - srow)
            denom = lax.dot_general(e, onehot, (((0,), (0,)), ((), ())),
                                    preferred_element_type=jnp.float32)
            drow = jnp.sum(onehot * denom, axis=1, keepdims=True)
            att = e / (drow + 1e-9)
            bagf = lax.dot_general(onehot, att * fea,
                                   (((0,), (0,)), ((), ())),
                                   preferred_element_type=jnp.float32)
            nrm = jnp.sqrt(jnp.sum(bagf * bagf, axis=1, keepdims=True))
            return att, bagf, bagf / (nrm + 1e-12)

        attq, bagfq, qn = branch(feaq_s[...], sq_s[...])
        attk, _, kn = branch(feak_s[...], sk_s[...])
        attq_ref[...] = attq
        attk_ref[...] = attk
        yprob_ref[...] = jax.nn.sigmoid(
            jnp.dot(bagfq, wcls_ref[...], preferred_element_type=jnp.float32))
        q_ref[...] = qn
        k_ref[...] = kn
        lpos_ref[...] = jnp.sum(qn * kn, axis=1, keepdims=True) / T


def _bank_body(q_ref, lpos_ref, lab_ref, bl_ref, bic_ref, bir_ref, bank_ref,
               logits_ref, nbank_ref, carry):
    j = pl.program_id(0)
    nb = K // COLS

    @pl.when(j < nb)
    def _():
        qm = q_ref[...]
        bank_t = bank_ref[...]
        ln = jnp.dot(qm, bank_t, preferred_element_type=jnp.float32)
        bl = bl_ref[0]
        mask = lab_ref[...] == bl
        ln = jnp.where(mask, -1e9, ln) / T
        head = jnp.where(j == 0, lpos_ref[...], carry[...])
        logits_ref[...] = jnp.concatenate([head, ln[:, :COLS - 1]], axis=1)
        carry[...] = ln[:, COLS - 1:COLS]
        bic = bic_ref[...]
        bir = bir_ref[...]
        ir = lax.broadcasted_iota(jnp.int32, (1, B), 1)
        ic = lax.broadcasted_iota(jnp.int32, (B, 1), 0)
        dup_later = (bic == bir) & (ir > ic)
        is_last = jnp.max(dup_later.astype(jnp.int32), axis=1,
                          keepdims=True) == 0
        cols = lax.broadcasted_iota(jnp.int32, (B, COLS), 1) + j * COLS
        sel = ((bic == cols) & is_last).astype(jnp.float32)
        hit = jnp.max(sel, axis=0, keepdims=True)
        over = lax.dot_general(qm, sel, (((0,), (0,)), ((), ())),
                               preferred_element_type=jnp.float32)
        nbank_ref[...] = bank_t * (1.0 - hit) + over

    @pl.when(j == nb)
    def _():
        logits_ref[:, 0:1] = carry[...]


def kernel(im_q, im_k, batch, bag_idx, label, bag_label, W_enc_q, W_self_q,
           V_q, U_q, w_att_q, W_cls_q, W_enc_k, W_self_k, bank):
    f32 = jnp.float32
    wq_cat = jnp.concatenate([W_enc_q, W_self_q], axis=1)
    wk_cat = jnp.concatenate([W_enc_k, W_self_k], axis=1)

    n_row_blocks = N_INST // ROWS
    (sfq, sfk, sq, sk, attq, attk, yprob, qn, kn, lpos) = pl.pallas_call(
        _enc_agg_body,
        grid=(n_row_blocks,),
        in_specs=[
            pl.BlockSpec((ROWS, D_IN), lambda i: (i, 0)),
            pl.BlockSpec((ROWS, D_IN), lambda i: (i, 0)),
            pl.BlockSpec((D_IN, 2 * DIM), lambda i: (0, 0)),
            pl.BlockSpec((D_IN, 2 * DIM), lambda i: (0, 0)),
            pl.BlockSpec((DIM, DIM), lambda i: (0, 0)),
            pl.BlockSpec((DIM, DIM), lambda i: (0, 0)),
            pl.BlockSpec((DIM, 1), lambda i: (0, 0)),
            pl.BlockSpec((N_INST, 1), lambda i: (0, 0)),
            pl.BlockSpec((DIM, 1), lambda i: (0, 0)),
        ],
        out_specs=[
            pl.BlockSpec((ROWS, DIM), lambda i: (i, 0)),
            pl.BlockSpec((ROWS, DIM), lambda i: (i, 0)),
            pl.BlockSpec((ROWS, 1), lambda i: (i, 0)),
            pl.BlockSpec((ROWS, 1), lambda i: (i, 0)),
            pl.BlockSpec((N_INST, 1), lambda i: (0, 0)),
            pl.BlockSpec((N_INST, 1), lambda i: (0, 0)),
            pl.BlockSpec((B, 1), lambda i: (0, 0)),
            pl.BlockSpec((B, DIM), lambda i: (0, 0)),
            pl.BlockSpec((B, DIM), lambda i: (0, 0)),
            pl.BlockSpec((B, 1), lambda i: (0, 0)),
        ],
        out_shape=[
            jax.ShapeDtypeStruct((N_INST, DIM), f32),
            jax.ShapeDtypeStruct((N_INST, DIM), f32),
            jax.ShapeDtypeStruct((N_INST, 1), f32),
            jax.ShapeDtypeStruct((N_INST, 1), f32),
            jax.ShapeDtypeStruct((N_INST, 1), f32),
            jax.ShapeDtypeStruct((N_INST, 1), f32),
            jax.ShapeDtypeStruct((B, 1), f32),
            jax.ShapeDtypeStruct((B, DIM), f32),
            jax.ShapeDtypeStruct((B, DIM), f32),
            jax.ShapeDtypeStruct((B, 1), f32),
        ],
        scratch_shapes=[
            pltpu.VMEM((N_INST, DIM), f32),
            pltpu.VMEM((N_INST, DIM), f32),
            pltpu.VMEM((N_INST, 1), f32),
            pltpu.VMEM((N_INST, 1), f32),
        ],
    )(im_q, im_k, wq_cat, wk_cat, V_q, U_q, w_att_q,
      batch.reshape(N_INST, 1).astype(jnp.int32), W_cls_q)

    n_col_blocks = K // COLS
    last = n_col_blocks - 1
    logits, nbank = pl.pallas_call(
        _bank_body,
        grid=(n_col_blocks + 1,),
        in_specs=[
            pl.BlockSpec((B, DIM), lambda j: (0, 0)),
            pl.BlockSpec((B, 1), lambda j: (0, 0)),
            pl.BlockSpec((B, 1), lambda j: (0, 0)),
            pl.BlockSpec((1, 1, COLS), lambda j: (jnp.minimum(j, last), 0, 0)),
            pl.BlockSpec((B, 1), lambda j: (0, 0)),
            pl.BlockSpec((1, B), lambda j: (0, 0)),
            pl.BlockSpec((DIM, COLS), lambda j: (0, jnp.minimum(j, last))),
        ],
        out_specs=[
            pl.BlockSpec((B, COLS), lambda j: (0, j)),
            pl.BlockSpec((DIM, COLS), lambda j: (0, jnp.minimum(j, last))),
        ],
        out_shape=[
            jax.ShapeDtypeStruct((B, K + 1), f32),
            jax.ShapeDtypeStruct((DIM, K), f32),
        ],
        scratch_shapes=[
            pltpu.VMEM((B, 1), f32),
        ],
    )(qn, lpos, label.reshape(B, 1).astype(jnp.int32),
      bag_label.reshape(n_col_blocks, 1, COLS).astype(jnp.int32),
      bag_idx.reshape(B, 1).astype(jnp.int32),
      bag_idx.reshape(1, B).astype(jnp.int32), bank)

    labels = jnp.zeros((B,), jnp.int32)
    return (yprob, logits, labels, nbank, sfq, sfk,
            attq.reshape(N_INST), attk.reshape(N_INST),
            sq.reshape(N_INST), sk.reshape(N_INST))
